# Initial kernel scaffold; baseline (speedup 1.0000x reference)
import jax
import jax.numpy as jnp
from jax.experimental import pallas as pl


def kernel(query, key, value):
    raise NotImplementedError("write your pallas kernel here")


# speedup vs baseline: 7.1157x; 7.1157x over previous
import functools
import math

import jax
import jax.numpy as jnp
from jax.experimental import pallas as pl
from jax.experimental.pallas import tpu as pltpu

_NUM_HEADS = 16
_HEAD_SIZE = 128
_TOPK_RATIO = 0.1

def _attn_kernel(q_ref, k_ref, v_ref, o_ref, *, k_keep, scale):
    q = q_ref[...]
    k = k_ref[...]
    v = v_ref[...]

    scores = jax.lax.dot_general(
        q, k, (((1,), (1,)), ((), ())), preferred_element_type=jnp.float32
    ) * scale

    b = jax.lax.bitcast_convert_type(scores, jnp.int32)
    int_min = jnp.int32(-(2**31))
    keys = jnp.where(b >= 0, b, int_min - b)

    lo0 = jnp.min(keys, axis=1, keepdims=True)
    hi0 = jnp.max(keys, axis=1, keepdims=True)

    def body(_, carry):
        lo, hi = carry
        mid = (lo >> 1) + (hi >> 1) + ((lo | hi) & 1)
        cnt = jnp.sum((keys >= mid).astype(jnp.int32), axis=1, keepdims=True)
        ge = cnt >= k_keep
        return jnp.where(ge, mid, lo), jnp.where(ge, hi, mid - 1)

    lo, hi = jax.lax.fori_loop(0, 32, body, (lo0, hi0))

    mask = keys >= lo

    m = jnp.max(scores, axis=1, keepdims=True)
    p = jnp.where(mask, jnp.exp(scores - m), 0.0)
    denom = jnp.sum(p, axis=1, keepdims=True)
    out = jax.lax.dot_general(
        p, v, (((1,), (0,)), ((), ())), preferred_element_type=jnp.float32
    )
    o_ref[...] = out / denom


@jax.jit
def kernel(query, key, value):
    B, S, HD = query.shape
    H, D = _NUM_HEADS, _HEAD_SIZE
    assert B == 1 and HD == H * D
    k_keep = max(1, int(_TOPK_RATIO * S))
    scale = 1.0 / math.sqrt(D)

    q2 = query.reshape(S, HD)
    k2 = key.reshape(S, HD)
    v2 = value.reshape(S, HD)

    QB = 256
    grid = (H, S // QB)

    out = pl.pallas_call(
        functools.partial(_attn_kernel, k_keep=k_keep, scale=scale),
        grid=grid,
        in_specs=[
            pl.BlockSpec((QB, D), lambda h, qi: (qi, h)),
            pl.BlockSpec((S, D), lambda h, qi: (0, h)),
            pl.BlockSpec((S, D), lambda h, qi: (0, h)),
        ],
        out_specs=pl.BlockSpec((QB, D), lambda h, qi: (qi, h)),
        out_shape=jax.ShapeDtypeStruct((S, HD), jnp.float32),
    )(q2, k2, v2)

    return out.reshape(B, S, HD)
